# baseline (speedup 1.0000x reference)
import jax
import jax.numpy as jnp
from jax import lax
from jax.experimental import pallas as pl
from jax.experimental.pallas import tpu as pltpu
from jax.experimental.pallas import tpu_sc as plsc

K = 12
NS = 500
SIGMA = 0.05
SEL = 2
D = 512
HID = 256
HD = 128
N = 198
DSP = 196
DP = 208

_NC = 2
_NSUB = 16
_NW = _NC * _NSUB
_UPT = 3
_NSPAD = 504
_DROW = 256
_HOFF = 248
_CHUNK = 64
_NCHUNK = 4
_NV = DP // 16
_ACC = K * DP


def _gelu(x):
    return 0.5 * x * (1.0 + lax.erf(x * 0.7071067811865476))


def _predictor_body(x_ref, nz_ref, lg_ref, lb_ref, wi_ref, wq_ref, wk_ref,
                    wv_ref, wo_ref, w1a_ref, w1b_ref, w2_ref, out_ref,
                    nzl_ref):
    nzl_ref[:NS, :DSP] = nz_ref[0]
    xb = x_ref[0]
    mu = jnp.mean(xb, axis=-1, keepdims=True)
    var = jnp.mean((xb - mu) ** 2, axis=-1, keepdims=True)
    ln = (xb - mu) * lax.rsqrt(var + 1e-5) * lg_ref[...] + lb_ref[...]
    h = _gelu(jnp.dot(ln, wi_ref[...], preferred_element_type=jnp.float32))
    q = jnp.dot(h, wq_ref[...], preferred_element_type=jnp.float32)
    k = jnp.dot(h, wk_ref[...], preferred_element_type=jnp.float32)
    v = jnp.dot(h, wv_ref[...], preferred_element_type=jnp.float32)
    heads = []
    for hh in range(2):
        qh = q[:, hh * HD:(hh + 1) * HD]
        kh = k[:, hh * HD:(hh + 1) * HD]
        vh = v[:, hh * HD:(hh + 1) * HD]
        s = lax.dot_general(qh, kh, (((1,), (1,)), ((), ())),
                            preferred_element_type=jnp.float32) * (HD ** -0.5)
        s = s - jnp.max(s, axis=-1, keepdims=True)
        e = jnp.exp(s)
        a = e / jnp.sum(e, axis=-1, keepdims=True)
        heads.append(jnp.dot(a, vh, preferred_element_type=jnp.float32))
    o = jnp.concatenate(heads, axis=-1)
    o = jnp.dot(o, wo_ref[...], preferred_element_type=jnp.float32)
    g = jnp.mean(o, axis=0, keepdims=True)
    u = (jnp.dot(o, w1a_ref[...], preferred_element_type=jnp.float32)
         + jnp.dot(g, w1b_ref[...], preferred_element_type=jnp.float32))
    u = _gelu(u)
    s = jnp.tanh(lax.dot_general(w2_ref[...], u, (((1,), (1,)), ((), ())),
                                 preferred_element_type=jnp.float32))
    sp = jnp.concatenate(
        [s[:, SEL:N], jnp.full((1, DP - DSP), -1e30, jnp.float32)], axis=1)
    out_ref[0] = sp


def _topk_sc_body(nz_hbm, sc_hbm, out_hbm, nz0, nz1, sp_buf, acc, sem0, sem1):
    wid = lax.axis_index("s") * _NC + lax.axis_index("c")
    lane = lax.iota(jnp.int32, 16)
    ones = jnp.ones((16,), jnp.float32)
    zeros16 = jnp.zeros((16,), jnp.float32)
    dvec = [lane + 16 * i for i in range(_NV)]

    for ui in range(_UPT):
        u = wid * _UPT + ui
        b = u // 2
        half = u % 2
        s0 = half * _HOFF
        lo = half * 256
        hi = 256 + half * (NS - 256)
        base_row = b * _NSPAD + s0
        pltpu.sync_copy(sc_hbm.at[b], sp_buf)
        spv = [sp_buf[pl.ds(16 * i, 16)] for i in range(_NV)]

        def _zero(i, carry):
            acc[pl.ds(i * 16, 16)] = zeros16
            return carry
        lax.fori_loop(0, _ACC // 16, _zero, 0)

        cp = pltpu.async_copy(nz_hbm.at[pl.ds(base_row, _CHUNK), :], nz0, sem0)
        for c in range(_NCHUNK):
            buf = nz0 if c % 2 == 0 else nz1
            ncp = None
            if c + 1 < _NCHUNK:
                nbuf = nz1 if c % 2 == 0 else nz0
                nsem = sem1 if c % 2 == 0 else sem0
                ncp = pltpu.async_copy(
                    nz_hbm.at[pl.ds(base_row + (c + 1) * _CHUNK, _CHUNK), :],
                    nbuf, nsem)
            cp.wait()
            sbase = s0 + c * _CHUNK

            def _one(r, buf, sbase):
                sid = sbase + r
                valid = (sid >= lo) & (sid < hi)
                level = []
                for i in range(_NV):
                    v = buf[r, pl.ds(i * 16, 16)]
                    pi = spv[i] + SIGMA * v
                    if i == _NV - 1:
                        pi = jnp.where(lane < 16 - (DP - DSP), pi, -3e38)
                    level.append(plsc.sort_key_val(pi, dvec[i]))
                while len(level) > 1:
                    nxt = []
                    for a in range(0, len(level) - 1, 2):
                        ak, av = level[a]
                        bk, bv = level[a + 1]
                        rk = lax.rev(bk, (0,))
                        rv = lax.rev(bv, (0,))
                        m = ak >= rk
                        nxt.append(plsc.sort_key_val(jnp.where(m, ak, rk),
                                                     jnp.where(m, av, rv)))
                    if len(level) % 2:
                        nxt.append(level[-1])
                    level = nxt
                tidx = jnp.where(lane < 16 - K, jnp.int32(DP), level[0][1])
                dsrt = jnp.sort(tidx)
                fidx = lane * DP + dsrt
                plsc.addupdate_scatter(acc, [fidx], ones,
                                       mask=(lane < K) & valid)

            def _pair(r2, carry, buf=buf, sbase=sbase):
                _one(2 * r2, buf, sbase)
                _one(2 * r2 + 1, buf, sbase)
                return carry
            lax.fori_loop(0, _CHUNK // 2, _pair, 0)
            if ncp is not None:
                cp = ncp
        pltpu.sync_copy(acc, out_hbm.at[u])


def _sel_body(part_ref, xs_ref, out_ref):
    ind = (part_ref[0, 0] + part_ref[0, 1]) * (1.0 / NS)
    xs = xs_ref[0][SEL:, :]
    out_ref[0] = lax.dot_general(ind[:, :DSP], xs, (((1,), (0,)), ((), ())),
                                 preferred_element_type=jnp.float32)


def kernel(x, noise, ln_gamma, ln_beta, W_in, Wq, Wk, Wv, Wo, W1, W2):
    B, L, Dd = x.shape
    xr = x.reshape(-1, N, Dd)
    nb = xr.shape[0]
    lg = ln_gamma.reshape(1, D)
    lb = ln_beta.reshape(1, D)
    W1T = W1.T

    full = lambda shp: pl.BlockSpec(shp, lambda i: tuple([0] * len(shp)))
    scores, nzl = pl.pallas_call(
        _predictor_body,
        grid=(nb,),
        in_specs=[
            pl.BlockSpec((1, N, D), lambda i: (i, 0, 0)),
            pl.BlockSpec((1, NS, DSP), lambda i: (i, 0, 0)),
            full((1, D)), full((1, D)), full((D, HID)),
            full((HID, HID)), full((HID, HID)), full((HID, HID)),
            full((HID, HID)), full((HID, HID)), full((HID, HID)),
            full((1, HID)),
        ],
        out_specs=[
            pl.BlockSpec((1, 1, DP), lambda i: (i, 0, 0)),
            pl.BlockSpec((_NSPAD, _DROW), lambda i: (i, 0)),
        ],
        out_shape=[
            jax.ShapeDtypeStruct((nb, 1, DP), jnp.float32),
            jax.ShapeDtypeStruct((nb * _NSPAD, _DROW), jnp.float32),
        ],
    )(xr, noise, lg, lb, W_in.T, Wq.T, Wk.T, Wv.T, Wo.T, W1T[:HID], W1T[HID:],
      W2.reshape(1, HID))

    mesh = plsc.VectorSubcoreMesh(core_axis_name="c", subcore_axis_name="s")
    topk_call = pl.kernel(
        _topk_sc_body,
        mesh=mesh,
        compiler_params=pltpu.CompilerParams(needs_layout_passes=False),
        out_type=jax.ShapeDtypeStruct((_NW * _UPT, _ACC), jnp.float32),
        scratch_types=[
            pltpu.VMEM((_CHUNK, _DROW), jnp.float32),
            pltpu.VMEM((_CHUNK, _DROW), jnp.float32),
            pltpu.VMEM((DP,), jnp.float32),
            pltpu.VMEM((_ACC,), jnp.float32),
            pltpu.SemaphoreType.DMA,
            pltpu.SemaphoreType.DMA,
        ],
    )
    partials = topk_call(nzl, scores.reshape(nb, DP))
    partials = partials.reshape(nb, 2, K, DP)

    selw = pl.pallas_call(
        _sel_body,
        grid=(nb,),
        in_specs=[
            pl.BlockSpec((1, 2, K, DP), lambda i: (i, 0, 0, 0)),
            pl.BlockSpec((1, N, D), lambda i: (i, 0, 0)),
        ],
        out_specs=pl.BlockSpec((1, K, D), lambda i: (i, 0, 0)),
        out_shape=jax.ShapeDtypeStruct((nb, K, D), jnp.float32),
    )(partials, xr)

    out = jnp.concatenate([xr[:, :SEL], selw], axis=1)
    return out.reshape(B, -1, Dd)


# speedup vs baseline: 1.1963x; 1.1963x over previous
import jax
import jax.numpy as jnp
from jax import lax
from jax.experimental import pallas as pl
from jax.experimental.pallas import tpu as pltpu
from jax.experimental.pallas import tpu_sc as plsc

K = 12
NS = 500
SIGMA = 0.05
SEL = 2
D = 512
HID = 256
HD = 128
N = 198
DSP = 196
DP = 208

_NC = 2
_NSUB = 16
_NW = _NC * _NSUB
_G = 24
_UPT = 3
_NSPAD = 512
_DROW = 256
_QROWS = 128
_CHUNK = 64
_NCHUNK = _QROWS // _CHUNK
_NV = DP // 16
_ACC = K * DP


def _gelu(x):
    return 0.5 * x * (1.0 + lax.erf(x * 0.7071067811865476))


def _predictor_body(x_ref, nz_ref, lg_ref, lb_ref, wi_ref, wq_ref, wk_ref,
                    wv_ref, wo_ref, w1a_ref, w1b_ref, w2_ref, out_ref,
                    nzl_ref):
    nzl_ref[:NS, :DSP] = nz_ref[0]
    xb = x_ref[0]
    mu = jnp.mean(xb, axis=-1, keepdims=True)
    var = jnp.mean((xb - mu) ** 2, axis=-1, keepdims=True)
    ln = (xb - mu) * lax.rsqrt(var + 1e-5) * lg_ref[...] + lb_ref[...]
    h = _gelu(jnp.dot(ln, wi_ref[...], preferred_element_type=jnp.float32))
    q = jnp.dot(h, wq_ref[...], preferred_element_type=jnp.float32)
    k = jnp.dot(h, wk_ref[...], preferred_element_type=jnp.float32)
    v = jnp.dot(h, wv_ref[...], preferred_element_type=jnp.float32)
    heads = []
    for hh in range(2):
        qh = q[:, hh * HD:(hh + 1) * HD]
        kh = k[:, hh * HD:(hh + 1) * HD]
        vh = v[:, hh * HD:(hh + 1) * HD]
        s = lax.dot_general(qh, kh, (((1,), (1,)), ((), ())),
                            preferred_element_type=jnp.float32) * (HD ** -0.5)
        s = s - jnp.max(s, axis=-1, keepdims=True)
        e = jnp.exp(s)
        a = e / jnp.sum(e, axis=-1, keepdims=True)
        heads.append(jnp.dot(a, vh, preferred_element_type=jnp.float32))
    o = jnp.concatenate(heads, axis=-1)
    o = jnp.dot(o, wo_ref[...], preferred_element_type=jnp.float32)
    g = jnp.mean(o, axis=0, keepdims=True)
    u = (jnp.dot(o, w1a_ref[...], preferred_element_type=jnp.float32)
         + jnp.dot(g, w1b_ref[...], preferred_element_type=jnp.float32))
    u = _gelu(u)
    s = jnp.tanh(lax.dot_general(w2_ref[...], u, (((1,), (1,)), ((), ())),
                                 preferred_element_type=jnp.float32))
    sp = jnp.concatenate(
        [s[:, SEL:N], jnp.full((1, DP - DSP), -1e30, jnp.float32)], axis=1)
    out_ref[0] = sp


def _topk_sc_body(nz_hbm, sc_hbm, out_hbm, nz0, nz1, sp_buf, acc, sem0, sem1):
    wid = lax.axis_index("s") * _NC + lax.axis_index("c")
    lane = lax.iota(jnp.int32, 16)
    ones = jnp.ones((16,), jnp.float32)
    zeros16 = jnp.zeros((16,), jnp.float32)
    dvec = [lane + 16 * i for i in range(_NV)]

    for ui in range(_UPT):
        u = wid * _UPT + ui
        b = u // 4
        s0 = (u % 4) * _QROWS
        base_row = b * _NSPAD + s0
        pltpu.sync_copy(sc_hbm.at[b], sp_buf)
        spv = [sp_buf[pl.ds(16 * i, 16)] for i in range(_NV)]

        def _zero(i, carry):
            acc[pl.ds(i * 16, 16)] = zeros16
            return carry
        lax.fori_loop(0, _ACC // 16, _zero, 0)

        cp = pltpu.async_copy(nz_hbm.at[pl.ds(base_row, _CHUNK), :], nz0, sem0)
        for c in range(_NCHUNK):
            buf = nz0 if c % 2 == 0 else nz1
            ncp = None
            if c + 1 < _NCHUNK:
                nbuf = nz1 if c % 2 == 0 else nz0
                nsem = sem1 if c % 2 == 0 else sem0
                ncp = pltpu.async_copy(
                    nz_hbm.at[pl.ds(base_row + (c + 1) * _CHUNK, _CHUNK), :],
                    nbuf, nsem)
            cp.wait()
            sbase = s0 + c * _CHUNK

            def _one(r, buf, sbase):
                valid = sbase + r < NS
                level = []
                for i in range(_NV):
                    v = buf[r, pl.ds(i * 16, 16)]
                    pi = spv[i] + SIGMA * v
                    if i == _NV - 1:
                        pi = jnp.where(lane < 16 - (DP - DSP), pi, -3e38)
                    level.append(plsc.sort_key_val(pi, dvec[i]))
                while len(level) > 1:
                    nxt = []
                    for a in range(0, len(level) - 1, 2):
                        ak, av = level[a]
                        bk, bv = level[a + 1]
                        rk = lax.rev(bk, (0,))
                        rv = lax.rev(bv, (0,))
                        m = ak >= rk
                        nxt.append(plsc.sort_key_val(jnp.where(m, ak, rk),
                                                     jnp.where(m, av, rv)))
                    if len(level) % 2:
                        nxt.append(level[-1])
                    level = nxt
                tidx = jnp.where(lane < 16 - K, jnp.int32(DP), level[0][1])
                dsrt = jnp.sort(tidx)
                fidx = lane * DP + dsrt
                plsc.addupdate_scatter(acc, [fidx], ones,
                                       mask=(lane < K) & valid)

            def _row(r, carry, buf=buf, sbase=sbase):
                _one(r, buf, sbase)
                return carry
            lax.fori_loop(0, _CHUNK, _row, 0)
            if ncp is not None:
                cp = ncp
        pltpu.sync_copy(acc, out_hbm.at[u])


def _sel_body(part_ref, xs_ref, out_ref):
    ind = (part_ref[0, 0] + part_ref[0, 1]
           + part_ref[0, 2] + part_ref[0, 3]) * (1.0 / NS)
    xs = xs_ref[0][SEL:, :]
    out_ref[0] = lax.dot_general(ind[:, :DSP], xs, (((1,), (0,)), ((), ())),
                                 preferred_element_type=jnp.float32)


def kernel(x, noise, ln_gamma, ln_beta, W_in, Wq, Wk, Wv, Wo, W1, W2):
    B, L, Dd = x.shape
    xr = x.reshape(-1, N, Dd)
    lg = ln_gamma.reshape(1, D)
    lb = ln_beta.reshape(1, D)
    W1T = W1.T
    weights = (lg, lb, W_in.T, Wq.T, Wk.T, Wv.T, Wo.T, W1T[:HID], W1T[HID:],
               W2.reshape(1, HID))

    full = lambda shp: pl.BlockSpec(shp, lambda i: tuple([0] * len(shp)))
    mesh = plsc.VectorSubcoreMesh(core_axis_name="c", subcore_axis_name="s")
    topk_call = pl.kernel(
        _topk_sc_body,
        mesh=mesh,
        compiler_params=pltpu.CompilerParams(needs_layout_passes=False),
        out_type=jax.ShapeDtypeStruct((_NW * _UPT, _ACC), jnp.float32),
        scratch_types=[
            pltpu.VMEM((_CHUNK, _DROW), jnp.float32),
            pltpu.VMEM((_CHUNK, _DROW), jnp.float32),
            pltpu.VMEM((DP,), jnp.float32),
            pltpu.VMEM((_ACC,), jnp.float32),
            pltpu.SemaphoreType.DMA,
            pltpu.SemaphoreType.DMA,
        ],
    )

    selws = []
    for off in (0, _G):
        scores, nzl = pl.pallas_call(
            _predictor_body,
            grid=(_G,),
            in_specs=[
                pl.BlockSpec((1, N, D), lambda i, off=off: (i + off, 0, 0)),
                pl.BlockSpec((1, NS, DSP), lambda i, off=off: (i + off, 0, 0)),
                full((1, D)), full((1, D)), full((D, HID)),
                full((HID, HID)), full((HID, HID)), full((HID, HID)),
                full((HID, HID)), full((HID, HID)), full((HID, HID)),
                full((1, HID)),
            ],
            out_specs=[
                pl.BlockSpec((1, 1, DP), lambda i: (i, 0, 0)),
                pl.BlockSpec((_NSPAD, _DROW), lambda i: (i, 0)),
            ],
            out_shape=[
                jax.ShapeDtypeStruct((_G, 1, DP), jnp.float32),
                jax.ShapeDtypeStruct((_G * _NSPAD, _DROW), jnp.float32),
            ],
        )(xr, noise, *weights)

        partials = topk_call(nzl, scores.reshape(_G, DP))
        partials = partials.reshape(_G, 4, K, DP)

        selws.append(pl.pallas_call(
            _sel_body,
            grid=(_G,),
            in_specs=[
                pl.BlockSpec((1, 4, K, DP), lambda i: (i, 0, 0, 0)),
                pl.BlockSpec((1, N, D), lambda i, off=off: (i + off, 0, 0)),
            ],
            out_specs=pl.BlockSpec((1, K, D), lambda i: (i, 0, 0)),
            out_shape=jax.ShapeDtypeStruct((_G, K, D), jnp.float32),
        )(partials, xr))

    selw = jnp.concatenate(selws, axis=0)
    out = jnp.concatenate([xr[:, :SEL], selw], axis=1)
    return out.reshape(B, -1, Dd)


# speedup vs baseline: 1.2142x; 1.0149x over previous
import jax
import jax.numpy as jnp
from jax import lax
from jax.experimental import pallas as pl
from jax.experimental.pallas import tpu as pltpu
from jax.experimental.pallas import tpu_sc as plsc

K = 12
NS = 500
SIGMA = 0.05
SEL = 2
D = 512
HID = 256
HD = 128
N = 198
DSP = 196
DP = 208

_NC = 2
_NSUB = 16
_NW = _NC * _NSUB
_G = 12
_UPT = 3
_NSPAD = 512
_DROW = 256
_QROWS = 64
_CHUNK = 64
_NCHUNK = _QROWS // _CHUNK
_NV = DP // 16
_ACC = K * DP


def _gelu(x):
    return 0.5 * x * (1.0 + lax.erf(x * 0.7071067811865476))


def _predictor_body(x_ref, nz_ref, lg_ref, lb_ref, wi_ref, wq_ref, wk_ref,
                    wv_ref, wo_ref, w1a_ref, w1b_ref, w2_ref, out_ref,
                    nzl_ref):
    nzl_ref[:NS, :DSP] = nz_ref[0]
    xb = x_ref[0]
    mu = jnp.mean(xb, axis=-1, keepdims=True)
    var = jnp.mean((xb - mu) ** 2, axis=-1, keepdims=True)
    ln = (xb - mu) * lax.rsqrt(var + 1e-5) * lg_ref[...] + lb_ref[...]
    h = _gelu(jnp.dot(ln, wi_ref[...], preferred_element_type=jnp.float32))
    q = jnp.dot(h, wq_ref[...], preferred_element_type=jnp.float32)
    k = jnp.dot(h, wk_ref[...], preferred_element_type=jnp.float32)
    v = jnp.dot(h, wv_ref[...], preferred_element_type=jnp.float32)
    heads = []
    for hh in range(2):
        qh = q[:, hh * HD:(hh + 1) * HD]
        kh = k[:, hh * HD:(hh + 1) * HD]
        vh = v[:, hh * HD:(hh + 1) * HD]
        s = lax.dot_general(qh, kh, (((1,), (1,)), ((), ())),
                            preferred_element_type=jnp.float32) * (HD ** -0.5)
        s = s - jnp.max(s, axis=-1, keepdims=True)
        e = jnp.exp(s)
        a = e / jnp.sum(e, axis=-1, keepdims=True)
        heads.append(jnp.dot(a, vh, preferred_element_type=jnp.float32))
    o = jnp.concatenate(heads, axis=-1)
    o = jnp.dot(o, wo_ref[...], preferred_element_type=jnp.float32)
    g = jnp.mean(o, axis=0, keepdims=True)
    u = (jnp.dot(o, w1a_ref[...], preferred_element_type=jnp.float32)
         + jnp.dot(g, w1b_ref[...], preferred_element_type=jnp.float32))
    u = _gelu(u)
    s = jnp.tanh(lax.dot_general(w2_ref[...], u, (((1,), (1,)), ((), ())),
                                 preferred_element_type=jnp.float32))
    sp = jnp.concatenate(
        [s[:, SEL:N], jnp.full((1, DP - DSP), -1e30, jnp.float32)], axis=1)
    out_ref[0] = sp


def _topk_sc_body(nz_hbm, sc_hbm, out_hbm, nz0, nz1, sp_buf, acc, sem0, sem1):
    wid = lax.axis_index("s") * _NC + lax.axis_index("c")
    lane = lax.iota(jnp.int32, 16)
    ones = jnp.ones((16,), jnp.float32)
    zeros16 = jnp.zeros((16,), jnp.float32)
    dvec = [lane + 16 * i for i in range(_NV)]

    for ui in range(_UPT):
        u = wid * _UPT + ui
        b = u // 8
        s0 = (u % 8) * _QROWS
        base_row = b * _NSPAD + s0
        pltpu.sync_copy(sc_hbm.at[b], sp_buf)
        spv = [sp_buf[pl.ds(16 * i, 16)] for i in range(_NV)]

        def _zero(i, carry):
            acc[pl.ds(i * 16, 16)] = zeros16
            return carry
        lax.fori_loop(0, _ACC // 16, _zero, 0)

        cp = pltpu.async_copy(nz_hbm.at[pl.ds(base_row, _CHUNK), :], nz0, sem0)
        for c in range(_NCHUNK):
            buf = nz0 if c % 2 == 0 else nz1
            ncp = None
            if c + 1 < _NCHUNK:
                nbuf = nz1 if c % 2 == 0 else nz0
                nsem = sem1 if c % 2 == 0 else sem0
                ncp = pltpu.async_copy(
                    nz_hbm.at[pl.ds(base_row + (c + 1) * _CHUNK, _CHUNK), :],
                    nbuf, nsem)
            cp.wait()
            sbase = s0 + c * _CHUNK

            def _one(r, buf, sbase):
                valid = sbase + r < NS
                level = []
                for i in range(_NV):
                    v = buf[r, pl.ds(i * 16, 16)]
                    pi = spv[i] + SIGMA * v
                    if i == _NV - 1:
                        pi = jnp.where(lane < 16 - (DP - DSP), pi, -3e38)
                    level.append(plsc.sort_key_val(pi, dvec[i]))
                while len(level) > 1:
                    nxt = []
                    for a in range(0, len(level) - 1, 2):
                        ak, av = level[a]
                        bk, bv = level[a + 1]
                        rk = lax.rev(bk, (0,))
                        rv = lax.rev(bv, (0,))
                        m = ak >= rk
                        nxt.append(plsc.sort_key_val(jnp.where(m, ak, rk),
                                                     jnp.where(m, av, rv)))
                    if len(level) % 2:
                        nxt.append(level[-1])
                    level = nxt
                tidx = jnp.where(lane < 16 - K, jnp.int32(DP), level[0][1])
                dsrt = jnp.sort(tidx)
                fidx = lane * DP + dsrt
                plsc.addupdate_scatter(acc, [fidx], ones,
                                       mask=(lane < K) & valid)

            def _row(r, carry, buf=buf, sbase=sbase):
                _one(r, buf, sbase)
                return carry
            lax.fori_loop(0, _CHUNK, _row, 0)
            if ncp is not None:
                cp = ncp
        pltpu.sync_copy(acc, out_hbm.at[u])


def _sel_body(part_ref, xs_ref, out_ref):
    ind = (part_ref[0, 0] + part_ref[0, 1] + part_ref[0, 2] + part_ref[0, 3]
           + part_ref[0, 4] + part_ref[0, 5] + part_ref[0, 6]
           + part_ref[0, 7]) * (1.0 / NS)
    xs = xs_ref[0][SEL:, :]
    out_ref[0] = lax.dot_general(ind[:, :DSP], xs, (((1,), (0,)), ((), ())),
                                 preferred_element_type=jnp.float32)


def kernel(x, noise, ln_gamma, ln_beta, W_in, Wq, Wk, Wv, Wo, W1, W2):
    B, L, Dd = x.shape
    xr = x.reshape(-1, N, Dd)
    lg = ln_gamma.reshape(1, D)
    lb = ln_beta.reshape(1, D)
    W1T = W1.T
    weights = (lg, lb, W_in.T, Wq.T, Wk.T, Wv.T, Wo.T, W1T[:HID], W1T[HID:],
               W2.reshape(1, HID))

    full = lambda shp: pl.BlockSpec(shp, lambda i: tuple([0] * len(shp)))
    mesh = plsc.VectorSubcoreMesh(core_axis_name="c", subcore_axis_name="s")
    topk_call = pl.kernel(
        _topk_sc_body,
        mesh=mesh,
        compiler_params=pltpu.CompilerParams(needs_layout_passes=False),
        out_type=jax.ShapeDtypeStruct((_NW * _UPT, _ACC), jnp.float32),
        scratch_types=[
            pltpu.VMEM((_CHUNK, _DROW), jnp.float32),
            pltpu.VMEM((_CHUNK, _DROW), jnp.float32),
            pltpu.VMEM((DP,), jnp.float32),
            pltpu.VMEM((_ACC,), jnp.float32),
            pltpu.SemaphoreType.DMA,
            pltpu.SemaphoreType.DMA,
        ],
    )

    selws = []
    for off in (0, _G, 2 * _G, 3 * _G):
        scores, nzl = pl.pallas_call(
            _predictor_body,
            grid=(_G,),
            in_specs=[
                pl.BlockSpec((1, N, D), lambda i, off=off: (i + off, 0, 0)),
                pl.BlockSpec((1, NS, DSP), lambda i, off=off: (i + off, 0, 0)),
                full((1, D)), full((1, D)), full((D, HID)),
                full((HID, HID)), full((HID, HID)), full((HID, HID)),
                full((HID, HID)), full((HID, HID)), full((HID, HID)),
                full((1, HID)),
            ],
            out_specs=[
                pl.BlockSpec((1, 1, DP), lambda i: (i, 0, 0)),
                pl.BlockSpec((_NSPAD, _DROW), lambda i: (i, 0)),
            ],
            out_shape=[
                jax.ShapeDtypeStruct((_G, 1, DP), jnp.float32),
                jax.ShapeDtypeStruct((_G * _NSPAD, _DROW), jnp.float32),
            ],
        )(xr, noise, *weights)

        partials = topk_call(nzl, scores.reshape(_G, DP))
        partials = partials.reshape(_G, 8, K, DP)

        selws.append(pl.pallas_call(
            _sel_body,
            grid=(_G,),
            in_specs=[
                pl.BlockSpec((1, 8, K, DP), lambda i: (i, 0, 0, 0)),
                pl.BlockSpec((1, N, D), lambda i, off=off: (i + off, 0, 0)),
            ],
            out_specs=pl.BlockSpec((1, K, D), lambda i: (i, 0, 0)),
            out_shape=jax.ShapeDtypeStruct((_G, K, D), jnp.float32),
        )(partials, xr))

    selw = jnp.concatenate(selws, axis=0)
    out = jnp.concatenate([xr[:, :SEL], selw], axis=1)
    return out.reshape(B, -1, Dd)
